# baseline (speedup 1.0000x reference)
import functools

import jax
import jax.numpy as jnp
from jax import lax
from jax.experimental import pallas as pl
from jax.experimental.pallas import tpu as pltpu
from jax.experimental.pallas import tpu_sc as plsc

EMB = 128
NV0, NV1, NV2 = 5, 6, 2
NT = NV0 * NV1 * NV2
N_E = 320000

NC, NS = 2, 16
NW = NC * NS
PER_W = N_E // NW
CHUNK = 200
NCHUNK = PER_W // CHUNK
GRPS = (128, 72)
NBUF = 3


def _table_body(w0_ref, w1_ref, w2_ref, t_ref):
    r = lax.broadcasted_iota(jnp.int32, (NT, 1), 0)
    oh0 = (r // (NV1 * NV2) == lax.broadcasted_iota(jnp.int32, (NT, NV0), 1))
    oh1 = ((r // NV2) % NV1 == lax.broadcasted_iota(jnp.int32, (NT, NV1), 1))
    oh2 = (r % NV2 == lax.broadcasted_iota(jnp.int32, (NT, NV2), 1))
    t = jnp.dot(oh0.astype(jnp.float32), w0_ref[...],
                preferred_element_type=jnp.float32)
    t = t + jnp.dot(oh1.astype(jnp.float32), w1_ref[...],
                    preferred_element_type=jnp.float32)
    t = t + jnp.dot(oh2.astype(jnp.float32), w2_ref[...],
                    preferred_element_type=jnp.float32)
    t_ref[...] = t


def _build_table(w0, w1, w2):
    return pl.pallas_call(
        _table_body,
        out_shape=jax.ShapeDtypeStruct((NT, EMB), jnp.float32),
    )(w0, w1, w2)


_mesh = plsc.VectorSubcoreMesh(core_axis_name="c", subcore_axis_name="s")


@functools.partial(
    pl.kernel,
    mesh=_mesh,
    out_type=jax.ShapeDtypeStruct((N_E, EMB), jnp.float32),
    scratch_types=[
        pltpu.VMEM_SHARED((NT, EMB), jnp.float32),
        pltpu.VMEM((PER_W,), jnp.int32),
        pltpu.VMEM((PER_W,), jnp.int32),
        pltpu.VMEM((PER_W,), jnp.int32),
        pltpu.VMEM((NBUF, CHUNK, EMB), jnp.float32),
        pltpu.SemaphoreType.DMA,
        pltpu.SemaphoreType.DMA,
        pltpu.SemaphoreType.DMA,
        pltpu.SemaphoreType.DMA,
        pltpu.SemaphoreType.DMA,
        pltpu.SemaphoreType.DMA,
    ],
)
def _gather_kernel(ea_hbm, t_hbm, out_hbm,
                   t_sh, codes, a1v, a2v, rows,
                   g0, g1, g2, s0, s1, s2):
    sid = lax.axis_index("s")
    wid = sid * NC + lax.axis_index("c")
    gsem = (g0, g1, g2)
    ssem = (s0, s1, s2)

    @pl.when(sid == 0)
    def _stage_table():
        pltpu.sync_copy(t_hbm, t_sh)

    base_w = wid * PER_W
    pltpu.sync_copy(ea_hbm.at[pl.ds(base_w, PER_W)], codes)
    pltpu.sync_copy(ea_hbm.at[pl.ds(N_E + base_w, PER_W)], a1v)
    pltpu.sync_copy(ea_hbm.at[pl.ds(2 * N_E + base_w, PER_W)], a2v)

    def code_body(i, carry):
        sl = pl.ds(pl.multiple_of(i * 16, 16), 16)
        codes[sl] = codes[sl] * (NV1 * NV2) + a1v[sl] * NV2 + a2v[sl]
        return carry

    lax.fori_loop(0, PER_W // 16, code_body, 0)

    plsc.subcore_barrier()

    def fire_gathers(g, b):
        off = 0
        for sz in GRPS:
            pltpu.async_copy(
                t_hbm.at[codes.at[pl.ds(g * CHUNK + off, sz)]],
                rows.at[b, pl.ds(off, sz)], gsem[b])
            off += sz

    def drain_gathers(b):
        pltpu.make_async_copy(out_hbm.at[pl.ds(0, CHUNK)], rows.at[b],
                              gsem[b]).wait()

    def fire_store(g, b):
        pltpu.async_copy(rows.at[b],
                         out_hbm.at[pl.ds(base_w + g * CHUNK, CHUNK)],
                         ssem[b])

    def drain_store(b):
        pltpu.make_async_copy(rows.at[b], out_hbm.at[pl.ds(0, CHUNK)],
                              ssem[b]).wait()

    def step(g, k, drain_prev, prep):
        drain_gathers(k)
        fire_store(g, k)
        if prep:
            if drain_prev:
                drain_store((k + 2) % 3)
            fire_gathers(g + 2, (k + 2) % 3)

    fire_gathers(0, 0)
    fire_gathers(1, 1)
    step(0, 0, False, True)
    step(1, 1, True, True)
    step(2, 2, True, True)

    def triple_body(p, carry):
        g = 3 * p
        step(g, 0, True, True)
        step(g + 1, 1, True, True)
        step(g + 2, 2, True, True)
        return carry

    lax.fori_loop(1, NCHUNK // 3 - 1, triple_body, 0)

    step(45, 0, True, True)
    step(46, 1, True, True)
    step(47, 2, True, True)
    step(48, 0, True, False)
    step(49, 1, False, False)
    drain_store(2)
    drain_store(0)
    drain_store(1)


def kernel(edge_attr, W0, W1, W2):
    t = _build_table(W0, W1, W2)
    ea_t = edge_attr.astype(jnp.int32).T.reshape(3 * N_E)
    return _gather_kernel(ea_t, t)


# speedup vs baseline: 17.9584x; 17.9584x over previous
import functools

import jax
import jax.numpy as jnp
from jax import lax
from jax.experimental import pallas as pl
from jax.experimental.pallas import tpu as pltpu
from jax.experimental.pallas import tpu_sc as plsc

EMB = 128
NV0, NV1, NV2 = 5, 6, 2
NT = NV0 * NV1 * NV2
N_E = 320000

NC, NS = 2, 16
NW = NC * NS
PER_W = N_E // NW
CHUNK = 200
NCHUNK = PER_W // CHUNK
GRPS = (128, 72)
NBUF = 3


def _table_body(w0_ref, w1_ref, w2_ref, t_ref):
    r = lax.broadcasted_iota(jnp.int32, (NT, 1), 0)
    oh0 = (r // (NV1 * NV2) == lax.broadcasted_iota(jnp.int32, (NT, NV0), 1))
    oh1 = ((r // NV2) % NV1 == lax.broadcasted_iota(jnp.int32, (NT, NV1), 1))
    oh2 = (r % NV2 == lax.broadcasted_iota(jnp.int32, (NT, NV2), 1))
    t = jnp.dot(oh0.astype(jnp.float32), w0_ref[...],
                preferred_element_type=jnp.float32)
    t = t + jnp.dot(oh1.astype(jnp.float32), w1_ref[...],
                    preferred_element_type=jnp.float32)
    t = t + jnp.dot(oh2.astype(jnp.float32), w2_ref[...],
                    preferred_element_type=jnp.float32)
    t_ref[...] = t


def _build_table(w0, w1, w2):
    return pl.pallas_call(
        _table_body,
        out_shape=jax.ShapeDtypeStruct((NT, EMB), jnp.float32),
    )(w0, w1, w2)


_mesh = plsc.VectorSubcoreMesh(core_axis_name="c", subcore_axis_name="s")


@functools.partial(
    pl.kernel,
    mesh=_mesh,
    out_type=jax.ShapeDtypeStruct((N_E, EMB), jnp.float32),
    scratch_types=[
        pltpu.VMEM_SHARED((NT, EMB), jnp.float32),
        pltpu.VMEM((PER_W,), jnp.int32),
        pltpu.VMEM((PER_W,), jnp.int32),
        pltpu.VMEM((PER_W,), jnp.int32),
        pltpu.VMEM((NBUF, CHUNK, EMB), jnp.float32),
        pltpu.SemaphoreType.DMA,
        pltpu.SemaphoreType.DMA,
        pltpu.SemaphoreType.DMA,
        pltpu.SemaphoreType.DMA,
        pltpu.SemaphoreType.DMA,
        pltpu.SemaphoreType.DMA,
    ],
)
def _gather_kernel(ea_hbm, t_hbm, out_hbm,
                   t_sh, codes, a1v, a2v, rows,
                   g0, g1, g2, s0, s1, s2):
    sid = lax.axis_index("s")
    wid = sid * NC + lax.axis_index("c")
    gsem = (g0, g1, g2)
    ssem = (s0, s1, s2)

    @pl.when(sid == 0)
    def _stage_table():
        pltpu.sync_copy(t_hbm, t_sh)

    base_w = wid * PER_W
    pltpu.sync_copy(ea_hbm.at[pl.ds(base_w, PER_W)], codes)
    pltpu.sync_copy(ea_hbm.at[pl.ds(N_E + base_w, PER_W)], a1v)
    pltpu.sync_copy(ea_hbm.at[pl.ds(2 * N_E + base_w, PER_W)], a2v)

    def code_body(i, carry):
        sl = pl.ds(pl.multiple_of(i * 16, 16), 16)
        codes[sl] = codes[sl] * (NV1 * NV2) + a1v[sl] * NV2 + a2v[sl]
        return carry

    lax.fori_loop(0, PER_W // 16, code_body, 0)

    plsc.subcore_barrier()

    def fire_gathers(g, b):
        off = 0
        for sz in GRPS:
            pltpu.async_copy(
                t_sh.at[codes.at[pl.ds(g * CHUNK + off, sz)]],
                rows.at[b, pl.ds(off, sz)], gsem[b])
            off += sz

    def drain_gathers(b):
        pltpu.make_async_copy(out_hbm.at[pl.ds(0, CHUNK)], rows.at[b],
                              gsem[b]).wait()

    def fire_store(g, b):
        pltpu.async_copy(rows.at[b],
                         out_hbm.at[pl.ds(base_w + g * CHUNK, CHUNK)],
                         ssem[b])

    def drain_store(b):
        pltpu.make_async_copy(rows.at[b], out_hbm.at[pl.ds(0, CHUNK)],
                              ssem[b]).wait()

    def step(g, k, drain_prev, prep):
        drain_gathers(k)
        fire_store(g, k)
        if prep:
            if drain_prev:
                drain_store((k + 2) % 3)
            fire_gathers(g + 2, (k + 2) % 3)

    fire_gathers(0, 0)
    fire_gathers(1, 1)
    step(0, 0, False, True)
    step(1, 1, True, True)
    step(2, 2, True, True)

    def triple_body(p, carry):
        g = 3 * p
        step(g, 0, True, True)
        step(g + 1, 1, True, True)
        step(g + 2, 2, True, True)
        return carry

    lax.fori_loop(1, NCHUNK // 3 - 1, triple_body, 0)

    step(45, 0, True, True)
    step(46, 1, True, True)
    step(47, 2, True, True)
    step(48, 0, True, False)
    step(49, 1, False, False)
    drain_store(2)
    drain_store(0)
    drain_store(1)


def kernel(edge_attr, W0, W1, W2):
    t = _build_table(W0, W1, W2)
    ea_t = edge_attr.astype(jnp.int32).T.reshape(3 * N_E)
    return _gather_kernel(ea_t, t)
